# baseline (speedup 1.0000x reference)
import jax
import jax.numpy as jnp
from jax import lax
from jax.experimental import pallas as pl
from jax.experimental.pallas import tpu as pltpu
from jax.experimental.pallas import tpu_sc as plsc

_NC = 2
_NS = 16
_NW = _NC * _NS
_LANES = 16
_CHUNK = 64


def _sc_add(x2d, pos_emb, seq_len, d_model):
    n_rows = x2d.shape[0]
    rows_per_w = n_rows // _NW
    seq_per_w = rows_per_w
    mesh = plsc.VectorSubcoreMesh(core_axis_name="c", subcore_axis_name="s")
    n_col_vecs = d_model // _LANES

    @pl.kernel(
        out_type=jax.ShapeDtypeStruct((n_rows, d_model), jnp.float32),
        mesh=mesh,
        scratch_types=[
            pltpu.VMEM((_CHUNK, d_model), jnp.float32),
            pltpu.VMEM((_CHUNK, d_model), jnp.float32),
        ],
    )
    def run(x_hbm, p_hbm, o_hbm, xbuf, pbuf):
        wid = lax.axis_index("s") * _NC + lax.axis_index("c")
        row_base = wid * rows_per_w
        seq_base = lax.rem(wid * seq_per_w, seq_len)

        @pl.loop(0, rows_per_w // _CHUNK)
        def _chunk(g):
            off = g * _CHUNK
            pltpu.sync_copy(x_hbm.at[pl.ds(row_base + off, _CHUNK)], xbuf)
            pltpu.sync_copy(p_hbm.at[pl.ds(seq_base + off, _CHUNK)], pbuf)

            @pl.loop(0, _CHUNK)
            def _row(r):
                for ci in range(n_col_vecs):
                    sl = pl.ds(ci * _LANES, _LANES)
                    xbuf[r, sl] = xbuf[r, sl] + pbuf[r, sl]

            pltpu.sync_copy(xbuf, o_hbm.at[pl.ds(row_base + off, _CHUNK)])

    return run(x2d, pos_emb)


def kernel(x, pos_emb):
    batch, seq_len, d_model = x.shape
    x2d = x.reshape(batch * seq_len, d_model)
    out = _sc_add(x2d, pos_emb[:seq_len], seq_len, d_model)
    return out.reshape(batch, seq_len, d_model)


# speedup vs baseline: 1.1866x; 1.1866x over previous
import jax
import jax.numpy as jnp
from jax import lax
from jax.experimental import pallas as pl
from jax.experimental.pallas import tpu as pltpu
from jax.experimental.pallas import tpu_sc as plsc

_NC = 2
_NS = 16
_NW = _NC * _NS
_LANES = 16
_CHUNK = 32


def _sc_add(x2d, pos_emb, seq_len, d_model):
    n_rows = x2d.shape[0]
    rows_per_w = n_rows // _NW
    n_chunks = rows_per_w // _CHUNK
    mesh = plsc.VectorSubcoreMesh(core_axis_name="c", subcore_axis_name="s")
    n_col_vecs = d_model // _LANES

    @pl.kernel(
        out_type=jax.ShapeDtypeStruct((n_rows, d_model), jnp.float32),
        mesh=mesh,
        scratch_types=[
            pltpu.VMEM((2, _CHUNK, d_model), jnp.float32),
            pltpu.VMEM((2, _CHUNK, d_model), jnp.float32),
            pltpu.SemaphoreType.DMA((2,)),
            pltpu.SemaphoreType.DMA((2,)),
            pltpu.SemaphoreType.DMA((2,)),
        ],
    )
    def run(x_hbm, p_hbm, o_hbm, xbuf, pbuf, sx, sp, so):
        wid = lax.axis_index("s") * _NC + lax.axis_index("c")
        row_base = wid * rows_per_w
        seq_base = lax.rem(wid * rows_per_w, seq_len)

        def start_in(g, slot):
            off = g * _CHUNK
            pltpu.async_copy(
                x_hbm.at[pl.ds(row_base + off, _CHUNK)], xbuf.at[slot], sx.at[slot])
            pltpu.async_copy(
                p_hbm.at[pl.ds(seq_base + off, _CHUNK)], pbuf.at[slot], sp.at[slot])

        start_in(0, 0)

        @pl.loop(0, n_chunks)
        def _g(g):
            slot = lax.rem(g, 2)
            nxt = lax.rem(g + 1, 2)

            @pl.when(g + 1 < n_chunks)
            def _prefetch():
                @pl.when(g >= 1)
                def _drain():
                    pltpu.make_async_copy(
                        xbuf.at[nxt], o_hbm.at[pl.ds(0, _CHUNK)], so.at[nxt]
                    ).wait()

                start_in(g + 1, nxt)

            pltpu.make_async_copy(
                x_hbm.at[pl.ds(0, _CHUNK)], xbuf.at[slot], sx.at[slot]).wait()
            pltpu.make_async_copy(
                p_hbm.at[pl.ds(0, _CHUNK)], pbuf.at[slot], sp.at[slot]).wait()

            @pl.loop(0, _CHUNK)
            def _row(r):
                for ci in range(n_col_vecs):
                    sl = pl.ds(ci * _LANES, _LANES)
                    xbuf[slot, r, sl] = xbuf[slot, r, sl] + pbuf[slot, r, sl]

            off = g * _CHUNK
            pltpu.async_copy(
                xbuf.at[slot], o_hbm.at[pl.ds(row_base + off, _CHUNK)], so.at[slot])

        last = (n_chunks - 1) % 2
        pltpu.make_async_copy(
            xbuf.at[last], o_hbm.at[pl.ds(0, _CHUNK)], so.at[last]
        ).wait()

    return run(x2d, pos_emb)


def kernel(x, pos_emb):
    batch, seq_len, d_model = x.shape
    x2d = x.reshape(batch * seq_len, d_model)
    out = _sc_add(x2d, pos_emb[:seq_len], seq_len, d_model)
    return out.reshape(batch, seq_len, d_model)
